# baseline (speedup 1.0000x reference)
import functools

import jax
import jax.numpy as jnp
from jax import lax
from jax.experimental import pallas as pl
from jax.experimental.pallas import tpu as pltpu
from jax.experimental.pallas import tpu_sc as plsc

N = 50000
E = 800000
D_IN = 128
H = 64
HH = H // 2
G = 64
EPS = 1e-5
NPAD = 50176
P4 = NPAD // 4
B4 = 512
NBLK = 25
NTILES = 16
RPT = NPAD // NTILES
EW = 2000
EPT = E // NTILES
NWIN = EPT // EW
CW = 400
NCWIN = EPT // CW
CPC = 5
NCHUNK = NCWIN // CPC
ZR = RPT // 8
XR = RPT // 4
NEG_INF = float("-inf")


def _fill_f32(ref, n, value):
    v = jnp.full((16,), value, jnp.float32)

    def body(r, _):
        ref[pl.ds(r * 16, 16)] = v
        return 0

    lax.fori_loop(0, n // 16, body, 0)


def _make_deg_kernel():
    mesh = plsc.VectorSubcoreMesh(core_axis_name="c", subcore_axis_name="s")

    @functools.partial(
        pl.kernel,
        mesh=mesh,
        compiler_params=pltpu.CompilerParams(use_tc_tiling_on_sc=False,
                                             needs_layout_passes=False),
        out_type=jax.ShapeDtypeStruct((NPAD, HH), jnp.float32),
        scratch_types=[
            pltpu.VMEM_SHARED((NPAD,), jnp.float32),
            pltpu.VMEM((EW,), jnp.int32),
            pltpu.VMEM((EW,), jnp.float32),
            pltpu.VMEM((RPT,), jnp.float32),
            pltpu.VMEM((XR, HH), jnp.float32),
        ],
    )
    def deg_kernel(dst_hbm, degx_hbm, acc, dstv, onesv, degv, expv):
        c = lax.axis_index("c")
        s = lax.axis_index("s")

        @pl.when(c == 0)
        def _():
            _fill_f32(degv, RPT, 0.0)
            _fill_f32(onesv, EW, 1.0)
            pltpu.sync_copy(degv.at[pl.ds(0, RPT // 2)],
                            acc.at[pl.ds(s * RPT, RPT // 2)])
            pltpu.sync_copy(degv.at[pl.ds(0, RPT // 2)],
                            acc.at[pl.ds(s * RPT + RPT // 2, RPT // 2)])
            plsc.subcore_barrier()

            def win(k, _):
                off = s * EPT + k * EW
                pltpu.sync_copy(dst_hbm.at[pl.ds(off, EW)], dstv)
                pltpu.sync_copy(onesv, acc.at[dstv], add=True)
                return 0

            lax.fori_loop(0, NWIN, win, 0)
            plsc.subcore_barrier()
            pltpu.sync_copy(acc.at[pl.ds(s * RPT, RPT)], degv)
            for chunk in range(4):
                def expand(r, _):
                    idx = lax.broadcast(chunk * XR + r, (16,))
                    d16 = plsc.load_gather(degv, [idx])
                    expv[r, pl.ds(0, 16)] = d16
                    expv[r, pl.ds(16, 16)] = d16
                    return 0

                lax.fori_loop(0, XR, expand, 0)
                pltpu.sync_copy(
                    expv, degx_hbm.at[pl.ds(s * RPT + chunk * XR, XR)])

    return deg_kernel


def _make_conv_kernel():
    mesh = plsc.VectorSubcoreMesh(core_axis_name="c", subcore_axis_name="s")

    @functools.partial(
        pl.kernel,
        mesh=mesh,
        compiler_params=pltpu.CompilerParams(use_tc_tiling_on_sc=False),
        out_type=[
            jax.ShapeDtypeStruct((NPAD, HH), jnp.float32),
            jax.ShapeDtypeStruct((NPAD, HH), jnp.float32),
        ],
        scratch_types=[
            pltpu.VMEM_SHARED((NPAD, HH), jnp.float32),
            pltpu.VMEM((CPC, CW), jnp.int32),
            pltpu.VMEM((CPC, CW), jnp.int32),
            pltpu.VMEM((CW, HH), jnp.float32),
            pltpu.VMEM((CW, HH), jnp.float32),
            pltpu.SemaphoreType.DMA,
            pltpu.SemaphoreType.DMA,
            pltpu.SemaphoreType.DMA,
            pltpu.SemaphoreType.DMA,
        ],
    )
    def conv_kernel(src_hbm, dst_hbm, ya_hbm, yb_hbm, sa_hbm, sb_hbm,
                    acc, srcc, dstc, rows0, rows1, semg0, semg1,
                    sems0, sems1):
        c = lax.axis_index("c")
        s = lax.axis_index("s")
        rows = (rows0, rows1)
        semg = (semg0, semg1)
        sems = (sems0, sems1)

        def init_acc(ytab):
            for j in range(8):
                off = s * RPT + j * ZR
                pltpu.sync_copy(ytab.at[pl.ds(off, ZR)], rows0.at[pl.ds(0, ZR)])
                pltpu.sync_copy(rows0.at[pl.ds(0, ZR)], acc.at[pl.ds(off, ZR)])

        def run(ytab, outtab):
            init_acc(ytab)
            plsc.subcore_barrier()
            rowbase = s * NCWIN

            def load_chunk(ch):
                pltpu.sync_copy(src_hbm.at[pl.ds(rowbase + ch * CPC, CPC)],
                                srcc)
                pltpu.sync_copy(dst_hbm.at[pl.ds(rowbase + ch * CPC, CPC)],
                                dstc)

            def gstart(m, p):
                pltpu.async_copy(ytab.at[srcc.at[m]], rows[p], semg[p])

            def gwait(m, p):
                pltpu.make_async_copy(ytab.at[srcc.at[m]], rows[p],
                                      semg[p]).wait()

            def sstart(m, p):
                pltpu.async_copy(rows[p], acc.at[dstc.at[m]], sems[p],
                                 add=True)

            def swait(m, p):
                pltpu.make_async_copy(rows[p], acc.at[dstc.at[m]],
                                      sems[p]).wait()

            def do_chunk(ch, parity, boundary):
                for m in range(CPC - 1):
                    p = (parity + m) % 2
                    gwait(m, p)
                    if m > 0:
                        swait(m - 1, 1 - p)
                    gstart(m + 1, 1 - p)
                    sstart(m, p)
                p4 = (parity + CPC - 1) % 2
                gwait(CPC - 1, p4)
                swait(CPC - 2, 1 - p4)
                pltpu.sync_copy(rows[p4], acc.at[dstc.at[CPC - 1]], add=True)
                if boundary:
                    load_chunk(ch + 1)
                    gstart(0, (parity + CPC) % 2)

            load_chunk(0)
            gstart(0, 0)

            def pair(j, _):
                do_chunk(2 * j, 0, True)
                do_chunk(2 * j + 1, 1, True)
                return 0

            lax.fori_loop(0, (NCHUNK - 1) // 2, pair, 0)
            do_chunk(NCHUNK - 1, 0, False)

            plsc.subcore_barrier()
            for j in range(8):
                off = s * RPT + j * ZR
                pltpu.sync_copy(acc.at[pl.ds(off, ZR)], rows0.at[pl.ds(0, ZR)])
                pltpu.sync_copy(rows0.at[pl.ds(0, ZR)],
                                outtab.at[pl.ds(off, ZR)])

        @pl.when(c == 0)
        def _():
            run(ya_hbm, sa_hbm)

        @pl.when(c == 1)
        def _():
            run(yb_hbm, sb_hbm)

    return conv_kernel


def _dot(a, b):
    return lax.dot_general(a, b, (((1,), (0,)), ((), ())),
                           preferred_element_type=jnp.float32)


def _node_mask(i):
    r = lax.broadcasted_iota(jnp.int32, (B4, 128), 0) + i * B4
    p = lax.broadcasted_iota(jnp.int32, (B4, 128), 1) // HH
    return r * 4 + p < N


def _mm1_body(x_ref, degx_ref, bdwe_ref, be4_ref, bdw1a_ref, bdw1b_ref,
              ya_ref, yb_ref):
    dinv = lax.rsqrt(degx_ref[...] + 1.0)
    h0 = jnp.maximum(_dot(x_ref[...], bdwe_ref[...]) + be4_ref[...], 0.0)
    ya_ref[...] = _dot(h0, bdw1a_ref[...]) * dinv
    yb_ref[...] = _dot(h0, bdw1b_ref[...]) * dinv


def _stats_body(sa_ref, sb_ref, degx_ref, ba4_ref, bb4_ref,
                ta_ref, tb_ref, suma_ref, sqa_ref, sumb_ref, sqb_ref):
    i = pl.program_id(0)
    dinv = lax.rsqrt(degx_ref[...] + 1.0)
    ta = sa_ref[...] * dinv + ba4_ref[...]
    tb = sb_ref[...] * dinv + bb4_ref[...]
    ta_ref[...] = ta
    tb_ref[...] = tb
    m = _node_mask(i)
    tam = jnp.where(m, ta, 0.0)
    tbm = jnp.where(m, tb, 0.0)

    @pl.when(i == 0)
    def _():
        suma_ref[...] = jnp.zeros_like(suma_ref)
        sqa_ref[...] = jnp.zeros_like(sqa_ref)
        sumb_ref[...] = jnp.zeros_like(sumb_ref)
        sqb_ref[...] = jnp.zeros_like(sqb_ref)

    suma_ref[...] += jnp.sum(tam, axis=0, keepdims=True)
    sqa_ref[...] += jnp.sum(tam * tam, axis=0, keepdims=True)
    sumb_ref[...] += jnp.sum(tbm, axis=0, keepdims=True)
    sqb_ref[...] += jnp.sum(tbm * tbm, axis=0, keepdims=True)


def _mm2_body(ta_ref, tb_ref, degx_ref, sca_ref, sha_ref, scb_ref, shb_ref,
              qaa_ref, qab_ref, qba_ref, qbb_ref, ya_ref, yb_ref):
    dinv = lax.rsqrt(degx_ref[...] + 1.0)
    ha = jnp.maximum(ta_ref[...] * sca_ref[...] + sha_ref[...], 0.0)
    hb = jnp.maximum(tb_ref[...] * scb_ref[...] + shb_ref[...], 0.0)
    ya_ref[...] = (_dot(ha, qaa_ref[...]) + _dot(hb, qba_ref[...])) * dinv
    yb_ref[...] = (_dot(ha, qab_ref[...]) + _dot(hb, qbb_ref[...])) * dinv


def _pool_body(ta_ref, tb_ref, batch_ref, sca_ref, sha_ref, scb_ref, shb_ref,
               ma_ref, mb_ref, su_a_ref, su_b_ref, cnt_ref):
    i = pl.program_id(0)

    @pl.when(i == 0)
    def _():
        ma_ref[...] = jnp.full_like(ma_ref, NEG_INF)
        mb_ref[...] = jnp.full_like(mb_ref, NEG_INF)
        su_a_ref[...] = jnp.zeros_like(su_a_ref)
        su_b_ref[...] = jnp.zeros_like(su_b_ref)
        cnt_ref[...] = jnp.zeros_like(cnt_ref)

    ha = jnp.maximum(ta_ref[...] * sca_ref[...] + sha_ref[...], 0.0)
    hb = jnp.maximum(tb_ref[...] * scb_ref[...] + shb_ref[...], 0.0)
    b = batch_ref[...]
    m = _node_mask(i)
    blo = jnp.min(b)
    bhi = jnp.max(b)

    def gbody(g, _):
        sel = (b == g) & m
        ma_ref[pl.ds(g, 1), :] = jnp.maximum(
            ma_ref[pl.ds(g, 1), :],
            jnp.max(jnp.where(sel, ha, NEG_INF), axis=0, keepdims=True))
        mb_ref[pl.ds(g, 1), :] = jnp.maximum(
            mb_ref[pl.ds(g, 1), :],
            jnp.max(jnp.where(sel, hb, NEG_INF), axis=0, keepdims=True))
        su_a_ref[pl.ds(g, 1), :] += jnp.sum(
            jnp.where(sel, ha, 0.0), axis=0, keepdims=True)
        su_b_ref[pl.ds(g, 1), :] += jnp.sum(
            jnp.where(sel, hb, 0.0), axis=0, keepdims=True)
        cnt_ref[pl.ds(g, 1), :] += jnp.sum(
            jnp.where(sel, 1.0, 0.0), axis=0, keepdims=True)
        return 0

    lax.fori_loop(blo, bhi + 1, gbody, 0)


def _final_body(ma_ref, mb_ref, sua_ref, sub_ref, cnt_ref, wout_ref, bout_ref,
                out_ref):
    def fold_max(x):
        return jnp.maximum(
            jnp.maximum(x[:, 0:HH], x[:, HH:2 * HH]),
            jnp.maximum(x[:, 2 * HH:3 * HH], x[:, 3 * HH:4 * HH]))

    def fold_sum(x):
        return (x[:, 0:HH] + x[:, HH:2 * HH]
                + x[:, 2 * HH:3 * HH] + x[:, 3 * HH:4 * HH])

    cnt = fold_sum(cnt_ref[...])
    denom = jnp.maximum(cnt, 1.0)
    mean_a = fold_sum(sua_ref[...]) / denom
    mean_b = fold_sum(sub_ref[...]) / denom
    comb = jnp.concatenate(
        [fold_max(ma_ref[...]), fold_max(mb_ref[...]), mean_a, mean_b],
        axis=1)
    out_ref[...] = _dot(comb, wout_ref[...]) + bout_ref[...]


def _pk_spec():
    return pl.BlockSpec((B4, 128), lambda i: (i, 0))


def _full_spec(r, w):
    return pl.BlockSpec((r, w), lambda i: (0, 0))


_mm1 = pl.pallas_call(
    _mm1_body,
    grid=(NBLK,),
    in_specs=[pl.BlockSpec((B4, 4 * D_IN), lambda i: (i, 0)), _pk_spec(),
              _full_spec(4 * D_IN, 4 * H), _full_spec(1, 4 * H),
              _full_spec(4 * H, 128), _full_spec(4 * H, 128)],
    out_specs=[_pk_spec(), _pk_spec()],
    out_shape=[jax.ShapeDtypeStruct((P4, 128), jnp.float32),
               jax.ShapeDtypeStruct((P4, 128), jnp.float32)],
)

_stats = pl.pallas_call(
    _stats_body,
    grid=(NBLK,),
    in_specs=[_pk_spec(), _pk_spec(), _pk_spec(),
              _full_spec(1, 128), _full_spec(1, 128)],
    out_specs=[_pk_spec(), _pk_spec(),
               _full_spec(1, 128), _full_spec(1, 128),
               _full_spec(1, 128), _full_spec(1, 128)],
    out_shape=[jax.ShapeDtypeStruct((P4, 128), jnp.float32),
               jax.ShapeDtypeStruct((P4, 128), jnp.float32),
               jax.ShapeDtypeStruct((1, 128), jnp.float32),
               jax.ShapeDtypeStruct((1, 128), jnp.float32),
               jax.ShapeDtypeStruct((1, 128), jnp.float32),
               jax.ShapeDtypeStruct((1, 128), jnp.float32)],
)

_mm2 = pl.pallas_call(
    _mm2_body,
    grid=(NBLK,),
    in_specs=[_pk_spec(), _pk_spec(), _pk_spec(),
              _full_spec(1, 128), _full_spec(1, 128),
              _full_spec(1, 128), _full_spec(1, 128),
              _full_spec(128, 128), _full_spec(128, 128),
              _full_spec(128, 128), _full_spec(128, 128)],
    out_specs=[_pk_spec(), _pk_spec()],
    out_shape=[jax.ShapeDtypeStruct((P4, 128), jnp.float32),
               jax.ShapeDtypeStruct((P4, 128), jnp.float32)],
)

_pool = pl.pallas_call(
    _pool_body,
    grid=(NBLK,),
    in_specs=[_pk_spec(), _pk_spec(), _pk_spec(),
              _full_spec(1, 128), _full_spec(1, 128),
              _full_spec(1, 128), _full_spec(1, 128)],
    out_specs=[_full_spec(G, 128)] * 5,
    out_shape=[jax.ShapeDtypeStruct((G, 128), jnp.float32)] * 5,
)

_final = pl.pallas_call(
    _final_body,
    grid=(1,),
    in_specs=[_full_spec(G, 128)] * 5 + [_full_spec(2 * H, 128),
                                         _full_spec(1, 128)],
    out_specs=_full_spec(G, 128),
    out_shape=jax.ShapeDtypeStruct((G, 128), jnp.float32),
)

_deg_kernel = _make_deg_kernel()
_conv_kernel = _make_conv_kernel()


def _blockdiag4(w):
    r, c = w.shape
    z = jnp.zeros((4 * r, 4 * c), jnp.float32)
    for k in range(4):
        z = z.at[k * r:(k + 1) * r, k * c:(k + 1) * c].set(w)
    return z


def _tile4(v):
    return jnp.tile(v.reshape(1, -1), (1, 4))


def _fold128(v):
    return jnp.sum(v.reshape(4, HH), axis=0)


def kernel(x, edge_index, batch, W_embed, b_embed, W1, b1, gamma1, beta1,
           W2, b2, gamma2, beta2, W_out, b_out):
    f32 = jnp.float32
    src = edge_index[0].astype(jnp.int32)
    dst = edge_index[1].astype(jnp.int32)
    src2 = src.reshape(E // CW, CW)
    dst2 = dst.reshape(E // CW, CW)
    batch_pad = jnp.pad(batch.astype(jnp.int32), (0, NPAD - N), mode="edge")
    batch4 = jnp.repeat(batch_pad, HH).reshape(P4, 128)
    x4 = jnp.concatenate(
        [x, jnp.zeros((NPAD - N, D_IN), f32)], axis=0).reshape(P4, 4 * D_IN)

    degx_lin = _deg_kernel(dst)
    degx = degx_lin.reshape(P4, 128)

    ya4, yb4 = _mm1(x4, degx, _blockdiag4(W_embed),
                    _tile4(b_embed), _blockdiag4(W1[:, :HH]),
                    _blockdiag4(W1[:, HH:]))

    def conv_bn(ya4_, yb4_, bvec, gamma, beta):
        sa, sb = _conv_kernel(src2, dst2,
                              ya4_.reshape(NPAD, HH), yb4_.reshape(NPAD, HH))
        ta4, tb4, sma, sqa, smb, sqb = _stats(
            sa.reshape(P4, 128), sb.reshape(P4, 128), degx,
            _tile4(bvec[:HH]), _tile4(bvec[HH:]))
        sm = jnp.concatenate([_fold128(sma), _fold128(smb)])
        sq = jnp.concatenate([_fold128(sqa), _fold128(sqb)])
        mu = sm / N
        var = sq / N - mu * mu
        scale = gamma / jnp.sqrt(var + EPS)
        shift = beta - mu * scale
        return ta4, tb4, scale, shift

    ta4, tb4, scale1, shift1 = conv_bn(ya4, yb4, b1, gamma1, beta1)

    y2a4, y2b4 = _mm2(ta4, tb4, degx,
                      _tile4(scale1[:HH]), _tile4(shift1[:HH]),
                      _tile4(scale1[HH:]), _tile4(shift1[HH:]),
                      _blockdiag4(W2[:HH, :HH]), _blockdiag4(W2[:HH, HH:]),
                      _blockdiag4(W2[HH:, :HH]), _blockdiag4(W2[HH:, HH:]))

    t2a4, t2b4, scale2, shift2 = conv_bn(y2a4, y2b4, b2, gamma2, beta2)

    ma, mb, sua, sub, cnt = _pool(t2a4, t2b4, batch4,
                                  _tile4(scale2[:HH]), _tile4(shift2[:HH]),
                                  _tile4(scale2[HH:]), _tile4(shift2[HH:]))

    return _final(ma, mb, sua, sub, cnt, W_out, b_out.reshape(1, 2 * H))


# speedup vs baseline: 1.0477x; 1.0477x over previous
import functools

import jax
import jax.numpy as jnp
from jax import lax
from jax.experimental import pallas as pl
from jax.experimental.pallas import tpu as pltpu
from jax.experimental.pallas import tpu_sc as plsc

N = 50000
E = 800000
D_IN = 128
H = 64
HH = H // 2
G = 64
EPS = 1e-5
NPAD = 50176
P4 = NPAD // 4
B4 = 512
NBLK = 25
NTILES = 16
RPT = NPAD // NTILES
EW = 2000
EPT = E // NTILES
NWIN = EPT // EW
CW = 400
NCWIN = EPT // CW
CPC = 5
NCHUNK = NCWIN // CPC
ZR = RPT // 8
XR = RPT // 4
NEG_INF = float("-inf")


def _fill_f32(ref, n, value):
    v = jnp.full((16,), value, jnp.float32)

    def body(r, _):
        ref[pl.ds(r * 16, 16)] = v
        return 0

    lax.fori_loop(0, n // 16, body, 0)


def _make_deg_kernel():
    mesh = plsc.VectorSubcoreMesh(core_axis_name="c", subcore_axis_name="s")

    @functools.partial(
        pl.kernel,
        mesh=mesh,
        compiler_params=pltpu.CompilerParams(use_tc_tiling_on_sc=False,
                                             needs_layout_passes=False),
        out_type=jax.ShapeDtypeStruct((NPAD, HH), jnp.float32),
        scratch_types=[
            pltpu.VMEM_SHARED((NPAD,), jnp.float32),
            pltpu.VMEM((EW,), jnp.int32),
            pltpu.VMEM((EW,), jnp.float32),
            pltpu.VMEM((RPT,), jnp.float32),
            pltpu.VMEM((XR, HH), jnp.float32),
        ],
    )
    def deg_kernel(dst_hbm, degx_hbm, acc, dstv, onesv, degv, expv):
        c = lax.axis_index("c")
        s = lax.axis_index("s")

        @pl.when(c == 0)
        def _():
            _fill_f32(degv, RPT, 0.0)
            _fill_f32(onesv, EW, 1.0)
            pltpu.sync_copy(degv.at[pl.ds(0, RPT // 2)],
                            acc.at[pl.ds(s * RPT, RPT // 2)])
            pltpu.sync_copy(degv.at[pl.ds(0, RPT // 2)],
                            acc.at[pl.ds(s * RPT + RPT // 2, RPT // 2)])
            plsc.subcore_barrier()

            def win(k, _):
                off = s * EPT + k * EW
                pltpu.sync_copy(dst_hbm.at[pl.ds(off, EW)], dstv)
                pltpu.sync_copy(onesv, acc.at[dstv], add=True)
                return 0

            lax.fori_loop(0, NWIN, win, 0)
            plsc.subcore_barrier()
            pltpu.sync_copy(acc.at[pl.ds(s * RPT, RPT)], degv)
            for chunk in range(4):
                def expand(r, _):
                    idx = lax.broadcast(chunk * XR + r, (16,))
                    d16 = plsc.load_gather(degv, [idx])
                    expv[r, pl.ds(0, 16)] = d16
                    expv[r, pl.ds(16, 16)] = d16
                    return 0

                lax.fori_loop(0, XR, expand, 0)
                pltpu.sync_copy(
                    expv, degx_hbm.at[pl.ds(s * RPT + chunk * XR, XR)])

    return deg_kernel


def _make_conv_kernel():
    mesh = plsc.VectorSubcoreMesh(core_axis_name="c", subcore_axis_name="s")

    @functools.partial(
        pl.kernel,
        mesh=mesh,
        compiler_params=pltpu.CompilerParams(use_tc_tiling_on_sc=False),
        out_type=[
            jax.ShapeDtypeStruct((NPAD, HH), jnp.float32),
            jax.ShapeDtypeStruct((NPAD, HH), jnp.float32),
        ],
        scratch_types=[
            pltpu.VMEM_SHARED((NPAD, HH), jnp.float32),
            pltpu.VMEM((CPC, CW), jnp.int32),
            pltpu.VMEM((CPC, CW), jnp.int32),
            pltpu.VMEM((CW, HH), jnp.float32),
            pltpu.VMEM((CW, HH), jnp.float32),
            pltpu.SemaphoreType.DMA,
            pltpu.SemaphoreType.DMA,
        ],
    )
    def conv_kernel(src_hbm, dst_hbm, ya_hbm, yb_hbm, sa_hbm, sb_hbm,
                    acc, srcc, dstc, rows0, rows1, semg0, semg1):
        c = lax.axis_index("c")
        s = lax.axis_index("s")
        rows = (rows0, rows1)
        semg = (semg0, semg1)

        def init_acc(ytab):
            for j in range(8):
                off = s * RPT + j * ZR
                pltpu.sync_copy(ytab.at[pl.ds(off, ZR)], rows0.at[pl.ds(0, ZR)])
                pltpu.sync_copy(rows0.at[pl.ds(0, ZR)], acc.at[pl.ds(off, ZR)])

        def run(ytab, outtab):
            init_acc(ytab)
            plsc.subcore_barrier()
            rowbase = s * NCWIN

            def load_chunk(ch):
                pltpu.sync_copy(src_hbm.at[pl.ds(rowbase + ch * CPC, CPC)],
                                srcc)
                pltpu.sync_copy(dst_hbm.at[pl.ds(rowbase + ch * CPC, CPC)],
                                dstc)

            def gstart(m, p):
                pltpu.async_copy(ytab.at[srcc.at[m]], rows[p], semg[p])

            def gwait(m, p):
                pltpu.make_async_copy(ytab.at[srcc.at[m]], rows[p],
                                      semg[p]).wait()

            def do_chunk(ch, parity, boundary):
                for m in range(CPC - 1):
                    p = (parity + m) % 2
                    gstart(m + 1, 1 - p)
                    gwait(m, p)
                    pltpu.sync_copy(rows[p], acc.at[dstc.at[m]], add=True)
                p4 = (parity + CPC - 1) % 2
                gwait(CPC - 1, p4)
                pltpu.sync_copy(rows[p4], acc.at[dstc.at[CPC - 1]], add=True)
                if boundary:
                    load_chunk(ch + 1)
                    gstart(0, (parity + CPC) % 2)

            load_chunk(0)
            gstart(0, 0)

            def pair(j, _):
                do_chunk(2 * j, 0, True)
                do_chunk(2 * j + 1, 1, True)
                return 0

            lax.fori_loop(0, (NCHUNK - 1) // 2, pair, 0)
            do_chunk(NCHUNK - 1, 0, False)

            plsc.subcore_barrier()
            for j in range(8):
                off = s * RPT + j * ZR
                pltpu.sync_copy(acc.at[pl.ds(off, ZR)], rows0.at[pl.ds(0, ZR)])
                pltpu.sync_copy(rows0.at[pl.ds(0, ZR)],
                                outtab.at[pl.ds(off, ZR)])

        @pl.when(c == 0)
        def _():
            run(ya_hbm, sa_hbm)

        @pl.when(c == 1)
        def _():
            run(yb_hbm, sb_hbm)

    return conv_kernel


def _dot(a, b):
    return lax.dot_general(a, b, (((1,), (0,)), ((), ())),
                           preferred_element_type=jnp.float32)


def _node_mask(i):
    r = lax.broadcasted_iota(jnp.int32, (B4, 128), 0) + i * B4
    p = lax.broadcasted_iota(jnp.int32, (B4, 128), 1) // HH
    return r * 4 + p < N


def _mm1_body(x_ref, degx_ref, bdwe_ref, be4_ref, bdw1a_ref, bdw1b_ref,
              ya_ref, yb_ref):
    dinv = lax.rsqrt(degx_ref[...] + 1.0)
    h0 = jnp.maximum(_dot(x_ref[...], bdwe_ref[...]) + be4_ref[...], 0.0)
    ya_ref[...] = _dot(h0, bdw1a_ref[...]) * dinv
    yb_ref[...] = _dot(h0, bdw1b_ref[...]) * dinv


def _stats_body(sa_ref, sb_ref, degx_ref, ba4_ref, bb4_ref,
                ta_ref, tb_ref, suma_ref, sqa_ref, sumb_ref, sqb_ref):
    i = pl.program_id(0)
    dinv = lax.rsqrt(degx_ref[...] + 1.0)
    ta = sa_ref[...] * dinv + ba4_ref[...]
    tb = sb_ref[...] * dinv + bb4_ref[...]
    ta_ref[...] = ta
    tb_ref[...] = tb
    m = _node_mask(i)
    tam = jnp.where(m, ta, 0.0)
    tbm = jnp.where(m, tb, 0.0)

    @pl.when(i == 0)
    def _():
        suma_ref[...] = jnp.zeros_like(suma_ref)
        sqa_ref[...] = jnp.zeros_like(sqa_ref)
        sumb_ref[...] = jnp.zeros_like(sumb_ref)
        sqb_ref[...] = jnp.zeros_like(sqb_ref)

    suma_ref[...] += jnp.sum(tam, axis=0, keepdims=True)
    sqa_ref[...] += jnp.sum(tam * tam, axis=0, keepdims=True)
    sumb_ref[...] += jnp.sum(tbm, axis=0, keepdims=True)
    sqb_ref[...] += jnp.sum(tbm * tbm, axis=0, keepdims=True)


def _mm2_body(ta_ref, tb_ref, degx_ref, sca_ref, sha_ref, scb_ref, shb_ref,
              qaa_ref, qab_ref, qba_ref, qbb_ref, ya_ref, yb_ref):
    dinv = lax.rsqrt(degx_ref[...] + 1.0)
    ha = jnp.maximum(ta_ref[...] * sca_ref[...] + sha_ref[...], 0.0)
    hb = jnp.maximum(tb_ref[...] * scb_ref[...] + shb_ref[...], 0.0)
    ya_ref[...] = (_dot(ha, qaa_ref[...]) + _dot(hb, qba_ref[...])) * dinv
    yb_ref[...] = (_dot(ha, qab_ref[...]) + _dot(hb, qbb_ref[...])) * dinv


def _pool_body(ta_ref, tb_ref, batch_ref, sca_ref, sha_ref, scb_ref, shb_ref,
               ma_ref, mb_ref, su_a_ref, su_b_ref, cnt_ref):
    i = pl.program_id(0)

    @pl.when(i == 0)
    def _():
        ma_ref[...] = jnp.full_like(ma_ref, NEG_INF)
        mb_ref[...] = jnp.full_like(mb_ref, NEG_INF)
        su_a_ref[...] = jnp.zeros_like(su_a_ref)
        su_b_ref[...] = jnp.zeros_like(su_b_ref)
        cnt_ref[...] = jnp.zeros_like(cnt_ref)

    ha = jnp.maximum(ta_ref[...] * sca_ref[...] + sha_ref[...], 0.0)
    hb = jnp.maximum(tb_ref[...] * scb_ref[...] + shb_ref[...], 0.0)
    b = batch_ref[...]
    m = _node_mask(i)
    blo = jnp.min(b)
    bhi = jnp.max(b)

    def gbody(g, _):
        sel = (b == g) & m
        ma_ref[pl.ds(g, 1), :] = jnp.maximum(
            ma_ref[pl.ds(g, 1), :],
            jnp.max(jnp.where(sel, ha, NEG_INF), axis=0, keepdims=True))
        mb_ref[pl.ds(g, 1), :] = jnp.maximum(
            mb_ref[pl.ds(g, 1), :],
            jnp.max(jnp.where(sel, hb, NEG_INF), axis=0, keepdims=True))
        su_a_ref[pl.ds(g, 1), :] += jnp.sum(
            jnp.where(sel, ha, 0.0), axis=0, keepdims=True)
        su_b_ref[pl.ds(g, 1), :] += jnp.sum(
            jnp.where(sel, hb, 0.0), axis=0, keepdims=True)
        cnt_ref[pl.ds(g, 1), :] += jnp.sum(
            jnp.where(sel, 1.0, 0.0), axis=0, keepdims=True)
        return 0

    lax.fori_loop(blo, bhi + 1, gbody, 0)


def _final_body(ma_ref, mb_ref, sua_ref, sub_ref, cnt_ref, wout_ref, bout_ref,
                out_ref):
    def fold_max(x):
        return jnp.maximum(
            jnp.maximum(x[:, 0:HH], x[:, HH:2 * HH]),
            jnp.maximum(x[:, 2 * HH:3 * HH], x[:, 3 * HH:4 * HH]))

    def fold_sum(x):
        return (x[:, 0:HH] + x[:, HH:2 * HH]
                + x[:, 2 * HH:3 * HH] + x[:, 3 * HH:4 * HH])

    cnt = fold_sum(cnt_ref[...])
    denom = jnp.maximum(cnt, 1.0)
    mean_a = fold_sum(sua_ref[...]) / denom
    mean_b = fold_sum(sub_ref[...]) / denom
    comb = jnp.concatenate(
        [fold_max(ma_ref[...]), fold_max(mb_ref[...]), mean_a, mean_b],
        axis=1)
    out_ref[...] = _dot(comb, wout_ref[...]) + bout_ref[...]


def _pk_spec():
    return pl.BlockSpec((B4, 128), lambda i: (i, 0))


def _full_spec(r, w):
    return pl.BlockSpec((r, w), lambda i: (0, 0))


_mm1 = pl.pallas_call(
    _mm1_body,
    grid=(NBLK,),
    in_specs=[pl.BlockSpec((B4, 4 * D_IN), lambda i: (i, 0)), _pk_spec(),
              _full_spec(4 * D_IN, 4 * H), _full_spec(1, 4 * H),
              _full_spec(4 * H, 128), _full_spec(4 * H, 128)],
    out_specs=[_pk_spec(), _pk_spec()],
    out_shape=[jax.ShapeDtypeStruct((P4, 128), jnp.float32),
               jax.ShapeDtypeStruct((P4, 128), jnp.float32)],
)

_stats = pl.pallas_call(
    _stats_body,
    grid=(NBLK,),
    in_specs=[_pk_spec(), _pk_spec(), _pk_spec(),
              _full_spec(1, 128), _full_spec(1, 128)],
    out_specs=[_pk_spec(), _pk_spec(),
               _full_spec(1, 128), _full_spec(1, 128),
               _full_spec(1, 128), _full_spec(1, 128)],
    out_shape=[jax.ShapeDtypeStruct((P4, 128), jnp.float32),
               jax.ShapeDtypeStruct((P4, 128), jnp.float32),
               jax.ShapeDtypeStruct((1, 128), jnp.float32),
               jax.ShapeDtypeStruct((1, 128), jnp.float32),
               jax.ShapeDtypeStruct((1, 128), jnp.float32),
               jax.ShapeDtypeStruct((1, 128), jnp.float32)],
)

_mm2 = pl.pallas_call(
    _mm2_body,
    grid=(NBLK,),
    in_specs=[_pk_spec(), _pk_spec(), _pk_spec(),
              _full_spec(1, 128), _full_spec(1, 128),
              _full_spec(1, 128), _full_spec(1, 128),
              _full_spec(128, 128), _full_spec(128, 128),
              _full_spec(128, 128), _full_spec(128, 128)],
    out_specs=[_pk_spec(), _pk_spec()],
    out_shape=[jax.ShapeDtypeStruct((P4, 128), jnp.float32),
               jax.ShapeDtypeStruct((P4, 128), jnp.float32)],
)

_pool = pl.pallas_call(
    _pool_body,
    grid=(NBLK,),
    in_specs=[_pk_spec(), _pk_spec(), _pk_spec(),
              _full_spec(1, 128), _full_spec(1, 128),
              _full_spec(1, 128), _full_spec(1, 128)],
    out_specs=[_full_spec(G, 128)] * 5,
    out_shape=[jax.ShapeDtypeStruct((G, 128), jnp.float32)] * 5,
)

_final = pl.pallas_call(
    _final_body,
    grid=(1,),
    in_specs=[_full_spec(G, 128)] * 5 + [_full_spec(2 * H, 128),
                                         _full_spec(1, 128)],
    out_specs=_full_spec(G, 128),
    out_shape=jax.ShapeDtypeStruct((G, 128), jnp.float32),
)

_deg_kernel = _make_deg_kernel()
_conv_kernel = _make_conv_kernel()


def _blockdiag4(w):
    r, c = w.shape
    z = jnp.zeros((4 * r, 4 * c), jnp.float32)
    for k in range(4):
        z = z.at[k * r:(k + 1) * r, k * c:(k + 1) * c].set(w)
    return z


def _tile4(v):
    return jnp.tile(v.reshape(1, -1), (1, 4))


def _fold128(v):
    return jnp.sum(v.reshape(4, HH), axis=0)


def kernel(x, edge_index, batch, W_embed, b_embed, W1, b1, gamma1, beta1,
           W2, b2, gamma2, beta2, W_out, b_out):
    f32 = jnp.float32
    src = edge_index[0].astype(jnp.int32)
    dst = edge_index[1].astype(jnp.int32)
    src2 = src.reshape(E // CW, CW)
    dst2 = dst.reshape(E // CW, CW)
    batch_pad = jnp.pad(batch.astype(jnp.int32), (0, NPAD - N), mode="edge")
    batch4 = jnp.repeat(batch_pad, HH).reshape(P4, 128)
    x4 = x.reshape(N // 4, 4 * D_IN)

    degx_lin = _deg_kernel(dst)
    degx = degx_lin.reshape(P4, 128)

    ya4, yb4 = _mm1(x4, degx, _blockdiag4(W_embed),
                    _tile4(b_embed), _blockdiag4(W1[:, :HH]),
                    _blockdiag4(W1[:, HH:]))

    def conv_bn(ya4_, yb4_, bvec, gamma, beta):
        sa, sb = _conv_kernel(src2, dst2,
                              ya4_.reshape(NPAD, HH), yb4_.reshape(NPAD, HH))
        ta4, tb4, sma, sqa, smb, sqb = _stats(
            sa.reshape(P4, 128), sb.reshape(P4, 128), degx,
            _tile4(bvec[:HH]), _tile4(bvec[HH:]))
        sm = jnp.concatenate([_fold128(sma), _fold128(smb)])
        sq = jnp.concatenate([_fold128(sqa), _fold128(sqb)])
        mu = sm / N
        var = sq / N - mu * mu
        scale = gamma / jnp.sqrt(var + EPS)
        shift = beta - mu * scale
        return ta4, tb4, scale, shift

    ta4, tb4, scale1, shift1 = conv_bn(ya4, yb4, b1, gamma1, beta1)

    y2a4, y2b4 = _mm2(ta4, tb4, degx,
                      _tile4(scale1[:HH]), _tile4(shift1[:HH]),
                      _tile4(scale1[HH:]), _tile4(shift1[HH:]),
                      _blockdiag4(W2[:HH, :HH]), _blockdiag4(W2[:HH, HH:]),
                      _blockdiag4(W2[HH:, :HH]), _blockdiag4(W2[HH:, HH:]))

    t2a4, t2b4, scale2, shift2 = conv_bn(y2a4, y2b4, b2, gamma2, beta2)

    ma, mb, sua, sub, cnt = _pool(t2a4, t2b4, batch4,
                                  _tile4(scale2[:HH]), _tile4(shift2[:HH]),
                                  _tile4(scale2[HH:]), _tile4(shift2[HH:]))

    return _final(ma, mb, sua, sub, cnt, W_out, b_out.reshape(1, 2 * H))


# speedup vs baseline: 1.0763x; 1.0273x over previous
import functools

import jax
import jax.numpy as jnp
from jax import lax
from jax.experimental import pallas as pl
from jax.experimental.pallas import tpu as pltpu
from jax.experimental.pallas import tpu_sc as plsc

N = 50000
E = 800000
D_IN = 128
H = 64
HH = H // 2
G = 64
EPS = 1e-5
NPAD = 50176
P4 = NPAD // 4
B4 = 512
NBLK = 25
NTILES = 16
RPT = NPAD // NTILES
EW = 2000
EPT = E // NTILES
NWIN = EPT // EW
CW = 400
NCWIN = EPT // CW
CPC = 5
NCHUNK = NCWIN // CPC
ZR = RPT // 8
XR = RPT // 4
NEG_INF = float("-inf")


def _fill_f32(ref, n, value):
    v = jnp.full((16,), value, jnp.float32)

    def body(r, _):
        ref[pl.ds(r * 16, 16)] = v
        return 0

    lax.fori_loop(0, n // 16, body, 0)


def _fill2d_f32(ref, rows, cols, value):
    v = jnp.full((16,), value, jnp.float32)

    def body(r, _):
        for j in range(cols // 16):
            ref[r, pl.ds(j * 16, 16)] = v
        return 0

    lax.fori_loop(0, rows, body, 0)


def _make_deg_kernel():
    mesh = plsc.VectorSubcoreMesh(core_axis_name="c", subcore_axis_name="s")

    @functools.partial(
        pl.kernel,
        mesh=mesh,
        compiler_params=pltpu.CompilerParams(use_tc_tiling_on_sc=False,
                                             needs_layout_passes=False),
        out_type=jax.ShapeDtypeStruct((NPAD, HH), jnp.float32),
        scratch_types=[
            pltpu.VMEM_SHARED((NPAD,), jnp.float32),
            pltpu.VMEM((CPC, CW), jnp.int32),
            pltpu.VMEM((CPC, CW), jnp.float32),
            pltpu.VMEM((RPT,), jnp.float32),
            pltpu.VMEM((XR, HH), jnp.float32),
        ],
    )
    def deg_kernel(edge_hbm, degx_hbm, acc, dstc, onesv, degv, expv):
        c = lax.axis_index("c")
        s = lax.axis_index("s")

        @pl.when(c == 0)
        def _():
            _fill_f32(degv, RPT, 0.0)
            _fill2d_f32(onesv, CPC, CW, 1.0)
            pltpu.sync_copy(degv.at[pl.ds(0, RPT // 2)],
                            acc.at[pl.ds(s * RPT, RPT // 2)])
            pltpu.sync_copy(degv.at[pl.ds(0, RPT // 2)],
                            acc.at[pl.ds(s * RPT + RPT // 2, RPT // 2)])
            plsc.subcore_barrier()
            rowbase = s * NCWIN

            def win(k, _):
                pltpu.sync_copy(
                    edge_hbm.at[1, pl.ds(rowbase + k * CPC, CPC)], dstc)
                for m in range(CPC):
                    pltpu.sync_copy(onesv.at[m], acc.at[dstc.at[m]], add=True)
                return 0

            lax.fori_loop(0, NCHUNK, win, 0)
            plsc.subcore_barrier()
            pltpu.sync_copy(acc.at[pl.ds(s * RPT, RPT)], degv)
            for chunk in range(4):
                def expand(r, _):
                    idx = lax.broadcast(chunk * XR + r, (16,))
                    d16 = plsc.load_gather(degv, [idx])
                    expv[r, pl.ds(0, 16)] = d16
                    expv[r, pl.ds(16, 16)] = d16
                    return 0

                lax.fori_loop(0, XR, expand, 0)
                pltpu.sync_copy(
                    expv, degx_hbm.at[pl.ds(s * RPT + chunk * XR, XR)])

    return deg_kernel


def _make_conv_kernel():
    mesh = plsc.VectorSubcoreMesh(core_axis_name="c", subcore_axis_name="s")

    @functools.partial(
        pl.kernel,
        mesh=mesh,
        compiler_params=pltpu.CompilerParams(use_tc_tiling_on_sc=False),
        out_type=[
            jax.ShapeDtypeStruct((NPAD, HH), jnp.float32),
            jax.ShapeDtypeStruct((NPAD, HH), jnp.float32),
        ],
        scratch_types=[
            pltpu.VMEM_SHARED((NPAD, HH), jnp.float32),
            pltpu.VMEM((CPC, CW), jnp.int32),
            pltpu.VMEM((CPC, CW), jnp.int32),
            pltpu.VMEM((CW, HH), jnp.float32),
            pltpu.VMEM((CW, HH), jnp.float32),
            pltpu.SemaphoreType.DMA,
            pltpu.SemaphoreType.DMA,
        ],
    )
    def conv_kernel(edge_hbm, ya_hbm, yb_hbm, sa_hbm, sb_hbm,
                    acc, srcc, dstc, rows0, rows1, semg0, semg1):
        c = lax.axis_index("c")
        s = lax.axis_index("s")
        rows = (rows0, rows1)
        semg = (semg0, semg1)

        def init_acc(ytab):
            for j in range(8):
                off = s * RPT + j * ZR
                pltpu.sync_copy(ytab.at[pl.ds(off, ZR)], rows0.at[pl.ds(0, ZR)])
                pltpu.sync_copy(rows0.at[pl.ds(0, ZR)], acc.at[pl.ds(off, ZR)])

        def run(ytab, outtab):
            init_acc(ytab)
            plsc.subcore_barrier()
            rowbase = s * NCWIN

            def load_chunk(ch):
                pltpu.sync_copy(
                    edge_hbm.at[0, pl.ds(rowbase + ch * CPC, CPC)], srcc)
                pltpu.sync_copy(
                    edge_hbm.at[1, pl.ds(rowbase + ch * CPC, CPC)], dstc)

            def gstart(m, p):
                pltpu.async_copy(ytab.at[srcc.at[m]], rows[p], semg[p])

            def gwait(m, p):
                pltpu.make_async_copy(ytab.at[srcc.at[m]], rows[p],
                                      semg[p]).wait()

            def do_chunk(ch, parity, boundary):
                for m in range(CPC - 1):
                    p = (parity + m) % 2
                    gstart(m + 1, 1 - p)
                    gwait(m, p)
                    pltpu.sync_copy(rows[p], acc.at[dstc.at[m]], add=True)
                p4 = (parity + CPC - 1) % 2
                gwait(CPC - 1, p4)
                pltpu.sync_copy(rows[p4], acc.at[dstc.at[CPC - 1]], add=True)
                if boundary:
                    load_chunk(ch + 1)
                    gstart(0, (parity + CPC) % 2)

            load_chunk(0)
            gstart(0, 0)

            def pair(j, _):
                do_chunk(2 * j, 0, True)
                do_chunk(2 * j + 1, 1, True)
                return 0

            lax.fori_loop(0, (NCHUNK - 1) // 2, pair, 0)
            do_chunk(NCHUNK - 1, 0, False)

            plsc.subcore_barrier()
            for j in range(8):
                off = s * RPT + j * ZR
                pltpu.sync_copy(acc.at[pl.ds(off, ZR)], rows0.at[pl.ds(0, ZR)])
                pltpu.sync_copy(rows0.at[pl.ds(0, ZR)],
                                outtab.at[pl.ds(off, ZR)])

        @pl.when(c == 0)
        def _():
            run(ya_hbm, sa_hbm)

        @pl.when(c == 1)
        def _():
            run(yb_hbm, sb_hbm)

    return conv_kernel


def _dot(a, b):
    return lax.dot_general(a, b, (((1,), (0,)), ((), ())),
                           preferred_element_type=jnp.float32)


def _node_mask(i):
    r = lax.broadcasted_iota(jnp.int32, (B4, 128), 0) + i * B4
    p = lax.broadcasted_iota(jnp.int32, (B4, 128), 1) // HH
    return r * 4 + p < N


def _mm1_body(x_ref, degx_ref, bdwe_ref, be4_ref, bdw1a_ref, bdw1b_ref,
              ya_ref, yb_ref):
    dinv = lax.rsqrt(degx_ref[...] + 1.0)
    h0 = jnp.maximum(_dot(x_ref[...], bdwe_ref[...]) + be4_ref[...], 0.0)
    ya_ref[...] = _dot(h0, bdw1a_ref[...]) * dinv
    yb_ref[...] = _dot(h0, bdw1b_ref[...]) * dinv


def _stats_body(sa_ref, sb_ref, degx_ref, ba4_ref, bb4_ref,
                ta_ref, tb_ref, suma_ref, sqa_ref, sumb_ref, sqb_ref):
    i = pl.program_id(0)
    dinv = lax.rsqrt(degx_ref[...] + 1.0)
    ta = sa_ref[...] * dinv + ba4_ref[...]
    tb = sb_ref[...] * dinv + bb4_ref[...]
    ta_ref[...] = ta
    tb_ref[...] = tb
    m = _node_mask(i)
    tam = jnp.where(m, ta, 0.0)
    tbm = jnp.where(m, tb, 0.0)

    @pl.when(i == 0)
    def _():
        suma_ref[...] = jnp.zeros_like(suma_ref)
        sqa_ref[...] = jnp.zeros_like(sqa_ref)
        sumb_ref[...] = jnp.zeros_like(sumb_ref)
        sqb_ref[...] = jnp.zeros_like(sqb_ref)

    suma_ref[...] += jnp.sum(tam, axis=0, keepdims=True)
    sqa_ref[...] += jnp.sum(tam * tam, axis=0, keepdims=True)
    sumb_ref[...] += jnp.sum(tbm, axis=0, keepdims=True)
    sqb_ref[...] += jnp.sum(tbm * tbm, axis=0, keepdims=True)


def _mm2_body(ta_ref, tb_ref, degx_ref, sca_ref, sha_ref, scb_ref, shb_ref,
              qaa_ref, qab_ref, qba_ref, qbb_ref, ya_ref, yb_ref):
    dinv = lax.rsqrt(degx_ref[...] + 1.0)
    ha = jnp.maximum(ta_ref[...] * sca_ref[...] + sha_ref[...], 0.0)
    hb = jnp.maximum(tb_ref[...] * scb_ref[...] + shb_ref[...], 0.0)
    ya_ref[...] = (_dot(ha, qaa_ref[...]) + _dot(hb, qba_ref[...])) * dinv
    yb_ref[...] = (_dot(ha, qab_ref[...]) + _dot(hb, qbb_ref[...])) * dinv


def _pool_body(ta_ref, tb_ref, batch_ref, sca_ref, sha_ref, scb_ref, shb_ref,
               ma_ref, mb_ref, su_a_ref, su_b_ref, cnt_ref):
    i = pl.program_id(0)

    @pl.when(i == 0)
    def _():
        ma_ref[...] = jnp.full_like(ma_ref, NEG_INF)
        mb_ref[...] = jnp.full_like(mb_ref, NEG_INF)
        su_a_ref[...] = jnp.zeros_like(su_a_ref)
        su_b_ref[...] = jnp.zeros_like(su_b_ref)
        cnt_ref[...] = jnp.zeros_like(cnt_ref)

    ha = jnp.maximum(ta_ref[...] * sca_ref[...] + sha_ref[...], 0.0)
    hb = jnp.maximum(tb_ref[...] * scb_ref[...] + shb_ref[...], 0.0)
    b = batch_ref[...]
    m = _node_mask(i)
    blo = jnp.min(b)
    bhi = jnp.max(b)

    def gbody(g, _):
        sel = (b == g) & m
        ma_ref[pl.ds(g, 1), :] = jnp.maximum(
            ma_ref[pl.ds(g, 1), :],
            jnp.max(jnp.where(sel, ha, NEG_INF), axis=0, keepdims=True))
        mb_ref[pl.ds(g, 1), :] = jnp.maximum(
            mb_ref[pl.ds(g, 1), :],
            jnp.max(jnp.where(sel, hb, NEG_INF), axis=0, keepdims=True))
        su_a_ref[pl.ds(g, 1), :] += jnp.sum(
            jnp.where(sel, ha, 0.0), axis=0, keepdims=True)
        su_b_ref[pl.ds(g, 1), :] += jnp.sum(
            jnp.where(sel, hb, 0.0), axis=0, keepdims=True)
        cnt_ref[pl.ds(g, 1), :] += jnp.sum(
            jnp.where(sel, 1.0, 0.0), axis=0, keepdims=True)
        return 0

    lax.fori_loop(blo, bhi + 1, gbody, 0)


def _final_body(ma_ref, mb_ref, sua_ref, sub_ref, cnt_ref, wout_ref, bout_ref,
                out_ref):
    def fold_max(x):
        return jnp.maximum(
            jnp.maximum(x[:, 0:HH], x[:, HH:2 * HH]),
            jnp.maximum(x[:, 2 * HH:3 * HH], x[:, 3 * HH:4 * HH]))

    def fold_sum(x):
        return (x[:, 0:HH] + x[:, HH:2 * HH]
                + x[:, 2 * HH:3 * HH] + x[:, 3 * HH:4 * HH])

    cnt = fold_sum(cnt_ref[...])
    denom = jnp.maximum(cnt, 1.0)
    mean_a = fold_sum(sua_ref[...]) / denom
    mean_b = fold_sum(sub_ref[...]) / denom
    comb = jnp.concatenate(
        [fold_max(ma_ref[...]), fold_max(mb_ref[...]), mean_a, mean_b],
        axis=1)
    out_ref[...] = _dot(comb, wout_ref[...]) + bout_ref[...]


def _pk_spec():
    return pl.BlockSpec((B4, 128), lambda i: (i, 0))


def _full_spec(r, w):
    return pl.BlockSpec((r, w), lambda i: (0, 0))


_mm1 = pl.pallas_call(
    _mm1_body,
    grid=(NBLK,),
    in_specs=[pl.BlockSpec((B4, 4 * D_IN), lambda i: (i, 0)), _pk_spec(),
              _full_spec(4 * D_IN, 4 * H), _full_spec(1, 4 * H),
              _full_spec(4 * H, 128), _full_spec(4 * H, 128)],
    out_specs=[_pk_spec(), _pk_spec()],
    out_shape=[jax.ShapeDtypeStruct((P4, 128), jnp.float32),
               jax.ShapeDtypeStruct((P4, 128), jnp.float32)],
)

_stats = pl.pallas_call(
    _stats_body,
    grid=(NBLK,),
    in_specs=[_pk_spec(), _pk_spec(), _pk_spec(),
              _full_spec(1, 128), _full_spec(1, 128)],
    out_specs=[_pk_spec(), _pk_spec(),
               _full_spec(1, 128), _full_spec(1, 128),
               _full_spec(1, 128), _full_spec(1, 128)],
    out_shape=[jax.ShapeDtypeStruct((P4, 128), jnp.float32),
               jax.ShapeDtypeStruct((P4, 128), jnp.float32),
               jax.ShapeDtypeStruct((1, 128), jnp.float32),
               jax.ShapeDtypeStruct((1, 128), jnp.float32),
               jax.ShapeDtypeStruct((1, 128), jnp.float32),
               jax.ShapeDtypeStruct((1, 128), jnp.float32)],
)

_mm2 = pl.pallas_call(
    _mm2_body,
    grid=(NBLK,),
    in_specs=[_pk_spec(), _pk_spec(), _pk_spec(),
              _full_spec(1, 128), _full_spec(1, 128),
              _full_spec(1, 128), _full_spec(1, 128),
              _full_spec(128, 128), _full_spec(128, 128),
              _full_spec(128, 128), _full_spec(128, 128)],
    out_specs=[_pk_spec(), _pk_spec()],
    out_shape=[jax.ShapeDtypeStruct((P4, 128), jnp.float32),
               jax.ShapeDtypeStruct((P4, 128), jnp.float32)],
)

_pool = pl.pallas_call(
    _pool_body,
    grid=(NBLK,),
    in_specs=[_pk_spec(), _pk_spec(), _pk_spec(),
              _full_spec(1, 128), _full_spec(1, 128),
              _full_spec(1, 128), _full_spec(1, 128)],
    out_specs=[_full_spec(G, 128)] * 5,
    out_shape=[jax.ShapeDtypeStruct((G, 128), jnp.float32)] * 5,
)

_final = pl.pallas_call(
    _final_body,
    grid=(1,),
    in_specs=[_full_spec(G, 128)] * 5 + [_full_spec(2 * H, 128),
                                         _full_spec(1, 128)],
    out_specs=_full_spec(G, 128),
    out_shape=jax.ShapeDtypeStruct((G, 128), jnp.float32),
)

_deg_kernel = _make_deg_kernel()
_conv_kernel = _make_conv_kernel()


def _blockdiag4(w):
    r, c = w.shape
    z = jnp.zeros((4 * r, 4 * c), jnp.float32)
    for k in range(4):
        z = z.at[k * r:(k + 1) * r, k * c:(k + 1) * c].set(w)
    return z


def _tile4(v):
    return jnp.tile(v.reshape(1, -1), (1, 4))


def _fold128(v):
    return jnp.sum(v.reshape(4, HH), axis=0)


def kernel(x, edge_index, batch, W_embed, b_embed, W1, b1, gamma1, beta1,
           W2, b2, gamma2, beta2, W_out, b_out):
    f32 = jnp.float32
    edge3 = edge_index.astype(jnp.int32).reshape(2, E // CW, CW)
    batch_pad = jnp.pad(batch.astype(jnp.int32), (0, NPAD - N), mode="edge")
    batch4 = jnp.repeat(batch_pad, HH).reshape(P4, 128)
    x4 = x.reshape(N // 4, 4 * D_IN)

    degx_lin = _deg_kernel(edge3)
    degx = degx_lin.reshape(P4, 128)

    ya4, yb4 = _mm1(x4, degx, _blockdiag4(W_embed),
                    _tile4(b_embed), _blockdiag4(W1[:, :HH]),
                    _blockdiag4(W1[:, HH:]))

    def conv_bn(ya4_, yb4_, bvec, gamma, beta):
        sa, sb = _conv_kernel(edge3,
                              ya4_.reshape(NPAD, HH), yb4_.reshape(NPAD, HH))
        ta4, tb4, sma, sqa, smb, sqb = _stats(
            sa.reshape(P4, 128), sb.reshape(P4, 128), degx,
            _tile4(bvec[:HH]), _tile4(bvec[HH:]))
        sm = jnp.concatenate([_fold128(sma), _fold128(smb)])
        sq = jnp.concatenate([_fold128(sqa), _fold128(sqb)])
        mu = sm / N
        var = sq / N - mu * mu
        scale = gamma / jnp.sqrt(var + EPS)
        shift = beta - mu * scale
        return ta4, tb4, scale, shift

    ta4, tb4, scale1, shift1 = conv_bn(ya4, yb4, b1, gamma1, beta1)

    y2a4, y2b4 = _mm2(ta4, tb4, degx,
                      _tile4(scale1[:HH]), _tile4(shift1[:HH]),
                      _tile4(scale1[HH:]), _tile4(shift1[HH:]),
                      _blockdiag4(W2[:HH, :HH]), _blockdiag4(W2[:HH, HH:]),
                      _blockdiag4(W2[HH:, :HH]), _blockdiag4(W2[HH:, HH:]))

    t2a4, t2b4, scale2, shift2 = conv_bn(y2a4, y2b4, b2, gamma2, beta2)

    ma, mb, sua, sub, cnt = _pool(t2a4, t2b4, batch4,
                                  _tile4(scale2[:HH]), _tile4(shift2[:HH]),
                                  _tile4(scale2[HH:]), _tile4(shift2[HH:]))

    return _final(ma, mb, sua, sub, cnt, W_out, b_out.reshape(1, 2 * H))


# speedup vs baseline: 1.1281x; 1.0481x over previous
import functools

import jax
import jax.numpy as jnp
from jax import lax
from jax.experimental import pallas as pl
from jax.experimental.pallas import tpu as pltpu
from jax.experimental.pallas import tpu_sc as plsc

N = 50000
E = 800000
D_IN = 128
H = 64
HH = H // 2
G = 64
EPS = 1e-5
NPAD = 50176
P4 = NPAD // 4
B4 = 512
NBLK = 25
NTILES = 16
RPT = NPAD // NTILES
EW = 2000
EPT = E // NTILES
NWIN = EPT // EW
CW = 400
NCWIN = EPT // CW
CPC = 5
NCHUNK = NCWIN // CPC
ZR = RPT // 8
XR = RPT // 4
NEG_INF = float("-inf")


def _fill_f32(ref, n, value):
    v = jnp.full((16,), value, jnp.float32)

    def body(r, _):
        ref[pl.ds(r * 16, 16)] = v
        return 0

    lax.fori_loop(0, n // 16, body, 0)


def _fill2d_f32(ref, rows, cols, value):
    v = jnp.full((16,), value, jnp.float32)

    def body(r, _):
        for j in range(cols // 16):
            ref[r, pl.ds(j * 16, 16)] = v
        return 0

    lax.fori_loop(0, rows, body, 0)


def _make_deg_kernel():
    mesh = plsc.VectorSubcoreMesh(core_axis_name="c", subcore_axis_name="s")

    @functools.partial(
        pl.kernel,
        mesh=mesh,
        compiler_params=pltpu.CompilerParams(use_tc_tiling_on_sc=False,
                                             needs_layout_passes=False),
        out_type=jax.ShapeDtypeStruct((NPAD, HH), jnp.float32),
        scratch_types=[
            pltpu.VMEM_SHARED((NPAD,), jnp.float32),
            pltpu.VMEM((CPC, CW), jnp.int32),
            pltpu.VMEM((CPC, CW), jnp.float32),
            pltpu.VMEM((RPT,), jnp.float32),
            pltpu.VMEM((XR, HH), jnp.float32),
        ],
    )
    def deg_kernel(edge_hbm, degx_hbm, acc, dstc, onesv, degv, expv):
        c = lax.axis_index("c")
        s = lax.axis_index("s")

        @pl.when(c == 0)
        def _():
            _fill_f32(degv, RPT, 0.0)
            _fill2d_f32(onesv, CPC, CW, 1.0)
            pltpu.sync_copy(degv.at[pl.ds(0, RPT // 2)],
                            acc.at[pl.ds(s * RPT, RPT // 2)])
            pltpu.sync_copy(degv.at[pl.ds(0, RPT // 2)],
                            acc.at[pl.ds(s * RPT + RPT // 2, RPT // 2)])
            plsc.subcore_barrier()
            rowbase = s * NCWIN

            def win(k, _):
                pltpu.sync_copy(
                    edge_hbm.at[1, pl.ds(rowbase + k * CPC, CPC)], dstc)
                for m in range(CPC):
                    pltpu.sync_copy(onesv.at[m], acc.at[dstc.at[m]], add=True)
                return 0

            lax.fori_loop(0, NCHUNK, win, 0)
            plsc.subcore_barrier()
            pltpu.sync_copy(acc.at[pl.ds(s * RPT, RPT)], degv)
            for chunk in range(4):
                def expand(r, _):
                    idx = lax.broadcast(chunk * XR + r, (16,))
                    d16 = plsc.load_gather(degv, [idx])
                    expv[r, pl.ds(0, 16)] = d16
                    expv[r, pl.ds(16, 16)] = d16
                    return 0

                lax.fori_loop(0, XR, expand, 0)
                pltpu.sync_copy(
                    expv, degx_hbm.at[pl.ds(s * RPT + chunk * XR, XR)])

    return deg_kernel


def _make_conv_kernel():
    mesh = plsc.VectorSubcoreMesh(core_axis_name="c", subcore_axis_name="s")

    @functools.partial(
        pl.kernel,
        mesh=mesh,
        compiler_params=pltpu.CompilerParams(use_tc_tiling_on_sc=False),
        out_type=[
            jax.ShapeDtypeStruct((NPAD, HH), jnp.float32),
            jax.ShapeDtypeStruct((NPAD, HH), jnp.float32),
        ],
        scratch_types=[
            pltpu.VMEM_SHARED((NPAD, HH), jnp.float32),
            pltpu.VMEM((CPC, CW), jnp.int32),
            pltpu.VMEM((CPC, CW), jnp.int32),
            pltpu.VMEM((CW, HH), jnp.float32),
            pltpu.VMEM((CW, HH), jnp.float32),
            pltpu.SemaphoreType.DMA,
            pltpu.SemaphoreType.DMA,
            pltpu.SemaphoreType.DMA,
            pltpu.SemaphoreType.DMA,
        ],
    )
    def conv_kernel(edge_hbm, ya_hbm, yb_hbm, sa_hbm, sb_hbm,
                    acc, srcc, dstc, rows0, rows1, semg0, semg1,
                    semi0, semi1):
        c = lax.axis_index("c")
        s = lax.axis_index("s")
        rows = (rows0, rows1)
        semg = (semg0, semg1)

        def init_acc(ytab):
            for j in range(8):
                off = s * RPT + j * ZR
                pltpu.sync_copy(ytab.at[pl.ds(off, ZR)], rows0.at[pl.ds(0, ZR)])
                pltpu.sync_copy(rows0.at[pl.ds(0, ZR)], acc.at[pl.ds(off, ZR)])

        def run(ytab, outtab):
            init_acc(ytab)
            plsc.subcore_barrier()
            rowbase = s * NCWIN

            def load_chunk(ch):
                pltpu.async_copy(
                    edge_hbm.at[0, pl.ds(rowbase + ch * CPC, CPC)], srcc,
                    semi0)
                pltpu.async_copy(
                    edge_hbm.at[1, pl.ds(rowbase + ch * CPC, CPC)], dstc,
                    semi1)
                pltpu.make_async_copy(
                    edge_hbm.at[0, pl.ds(rowbase + ch * CPC, CPC)], srcc,
                    semi0).wait()
                pltpu.make_async_copy(
                    edge_hbm.at[1, pl.ds(rowbase + ch * CPC, CPC)], dstc,
                    semi1).wait()

            def gstart(m, p):
                pltpu.async_copy(ytab.at[srcc.at[m]], rows[p], semg[p])

            def gwait(m, p):
                pltpu.make_async_copy(ytab.at[srcc.at[m]], rows[p],
                                      semg[p]).wait()

            def do_chunk(ch, parity, boundary):
                for m in range(CPC - 1):
                    p = (parity + m) % 2
                    gstart(m + 1, 1 - p)
                    gwait(m, p)
                    pltpu.sync_copy(rows[p], acc.at[dstc.at[m]], add=True)
                p4 = (parity + CPC - 1) % 2
                gwait(CPC - 1, p4)
                pltpu.sync_copy(rows[p4], acc.at[dstc.at[CPC - 1]], add=True)
                if boundary:
                    load_chunk(ch + 1)
                    gstart(0, (parity + CPC) % 2)

            load_chunk(0)
            gstart(0, 0)

            def pair(j, _):
                do_chunk(2 * j, 0, True)
                do_chunk(2 * j + 1, 1, True)
                return 0

            lax.fori_loop(0, (NCHUNK - 1) // 2, pair, 0)
            do_chunk(NCHUNK - 1, 0, False)

            plsc.subcore_barrier()
            for j in range(8):
                off = s * RPT + j * ZR
                pltpu.sync_copy(acc.at[pl.ds(off, ZR)], rows0.at[pl.ds(0, ZR)])
                pltpu.sync_copy(rows0.at[pl.ds(0, ZR)],
                                outtab.at[pl.ds(off, ZR)])

        @pl.when(c == 0)
        def _():
            run(ya_hbm, sa_hbm)

        @pl.when(c == 1)
        def _():
            run(yb_hbm, sb_hbm)

    return conv_kernel


def _dot(a, b):
    return lax.dot_general(a, b, (((1,), (0,)), ((), ())),
                           preferred_element_type=jnp.float32)


def _node_mask(i):
    r = lax.broadcasted_iota(jnp.int32, (B4, 128), 0) + i * B4
    p = lax.broadcasted_iota(jnp.int32, (B4, 128), 1) // HH
    return r * 4 + p < N


def _mm1_body(x_ref, degx_ref, bdwe_ref, be4_ref, bdw1a_ref, bdw1b_ref,
              ya_ref, yb_ref):
    dinv = lax.rsqrt(degx_ref[...] + 1.0)
    h0 = jnp.maximum(_dot(x_ref[...], bdwe_ref[...]) + be4_ref[...], 0.0)
    ya_ref[...] = _dot(h0, bdw1a_ref[...]) * dinv
    yb_ref[...] = _dot(h0, bdw1b_ref[...]) * dinv


def _stats_body(sa_ref, sb_ref, degx_ref, ba4_ref, bb4_ref,
                ta_ref, tb_ref, suma_ref, sqa_ref, sumb_ref, sqb_ref):
    i = pl.program_id(0)
    dinv = lax.rsqrt(degx_ref[...] + 1.0)
    ta = sa_ref[...] * dinv + ba4_ref[...]
    tb = sb_ref[...] * dinv + bb4_ref[...]
    ta_ref[...] = ta
    tb_ref[...] = tb
    m = _node_mask(i)
    tam = jnp.where(m, ta, 0.0)
    tbm = jnp.where(m, tb, 0.0)

    @pl.when(i == 0)
    def _():
        suma_ref[...] = jnp.zeros_like(suma_ref)
        sqa_ref[...] = jnp.zeros_like(sqa_ref)
        sumb_ref[...] = jnp.zeros_like(sumb_ref)
        sqb_ref[...] = jnp.zeros_like(sqb_ref)

    suma_ref[...] += jnp.sum(tam, axis=0, keepdims=True)
    sqa_ref[...] += jnp.sum(tam * tam, axis=0, keepdims=True)
    sumb_ref[...] += jnp.sum(tbm, axis=0, keepdims=True)
    sqb_ref[...] += jnp.sum(tbm * tbm, axis=0, keepdims=True)


def _mm2_body(ta_ref, tb_ref, degx_ref, sca_ref, sha_ref, scb_ref, shb_ref,
              qaa_ref, qab_ref, qba_ref, qbb_ref, ya_ref, yb_ref):
    dinv = lax.rsqrt(degx_ref[...] + 1.0)
    ha = jnp.maximum(ta_ref[...] * sca_ref[...] + sha_ref[...], 0.0)
    hb = jnp.maximum(tb_ref[...] * scb_ref[...] + shb_ref[...], 0.0)
    ya_ref[...] = (_dot(ha, qaa_ref[...]) + _dot(hb, qba_ref[...])) * dinv
    yb_ref[...] = (_dot(ha, qab_ref[...]) + _dot(hb, qbb_ref[...])) * dinv


def _pool_body(ta_ref, tb_ref, batch_ref, sca_ref, sha_ref, scb_ref, shb_ref,
               ma_ref, mb_ref, su_a_ref, su_b_ref, cnt_ref):
    i = pl.program_id(0)

    @pl.when(i == 0)
    def _():
        ma_ref[...] = jnp.full_like(ma_ref, NEG_INF)
        mb_ref[...] = jnp.full_like(mb_ref, NEG_INF)
        su_a_ref[...] = jnp.zeros_like(su_a_ref)
        su_b_ref[...] = jnp.zeros_like(su_b_ref)
        cnt_ref[...] = jnp.zeros_like(cnt_ref)

    ha = jnp.maximum(ta_ref[...] * sca_ref[...] + sha_ref[...], 0.0)
    hb = jnp.maximum(tb_ref[...] * scb_ref[...] + shb_ref[...], 0.0)
    b = batch_ref[...]
    m = _node_mask(i)
    blo = jnp.min(b)
    bhi = jnp.max(b)

    def gbody(g, _):
        sel = (b == g) & m
        ma_ref[pl.ds(g, 1), :] = jnp.maximum(
            ma_ref[pl.ds(g, 1), :],
            jnp.max(jnp.where(sel, ha, NEG_INF), axis=0, keepdims=True))
        mb_ref[pl.ds(g, 1), :] = jnp.maximum(
            mb_ref[pl.ds(g, 1), :],
            jnp.max(jnp.where(sel, hb, NEG_INF), axis=0, keepdims=True))
        su_a_ref[pl.ds(g, 1), :] += jnp.sum(
            jnp.where(sel, ha, 0.0), axis=0, keepdims=True)
        su_b_ref[pl.ds(g, 1), :] += jnp.sum(
            jnp.where(sel, hb, 0.0), axis=0, keepdims=True)
        cnt_ref[pl.ds(g, 1), :] += jnp.sum(
            jnp.where(sel, 1.0, 0.0), axis=0, keepdims=True)
        return 0

    lax.fori_loop(blo, bhi + 1, gbody, 0)


def _final_body(ma_ref, mb_ref, sua_ref, sub_ref, cnt_ref, wout_ref, bout_ref,
                out_ref):
    def fold_max(x):
        return jnp.maximum(
            jnp.maximum(x[:, 0:HH], x[:, HH:2 * HH]),
            jnp.maximum(x[:, 2 * HH:3 * HH], x[:, 3 * HH:4 * HH]))

    def fold_sum(x):
        return (x[:, 0:HH] + x[:, HH:2 * HH]
                + x[:, 2 * HH:3 * HH] + x[:, 3 * HH:4 * HH])

    cnt = fold_sum(cnt_ref[...])
    denom = jnp.maximum(cnt, 1.0)
    mean_a = fold_sum(sua_ref[...]) / denom
    mean_b = fold_sum(sub_ref[...]) / denom
    comb = jnp.concatenate(
        [fold_max(ma_ref[...]), fold_max(mb_ref[...]), mean_a, mean_b],
        axis=1)
    out_ref[...] = _dot(comb, wout_ref[...]) + bout_ref[...]


def _pk_spec():
    return pl.BlockSpec((B4, 128), lambda i: (i, 0))


def _full_spec(r, w):
    return pl.BlockSpec((r, w), lambda i: (0, 0))


_mm1 = pl.pallas_call(
    _mm1_body,
    grid=(NBLK,),
    in_specs=[pl.BlockSpec((B4, 4 * D_IN), lambda i: (i, 0)), _pk_spec(),
              _full_spec(4 * D_IN, 4 * H), _full_spec(1, 4 * H),
              _full_spec(4 * H, 128), _full_spec(4 * H, 128)],
    out_specs=[_pk_spec(), _pk_spec()],
    out_shape=[jax.ShapeDtypeStruct((P4, 128), jnp.float32),
               jax.ShapeDtypeStruct((P4, 128), jnp.float32)],
)

_stats = pl.pallas_call(
    _stats_body,
    grid=(NBLK,),
    in_specs=[_pk_spec(), _pk_spec(), _pk_spec(),
              _full_spec(1, 128), _full_spec(1, 128)],
    out_specs=[_pk_spec(), _pk_spec(),
               _full_spec(1, 128), _full_spec(1, 128),
               _full_spec(1, 128), _full_spec(1, 128)],
    out_shape=[jax.ShapeDtypeStruct((P4, 128), jnp.float32),
               jax.ShapeDtypeStruct((P4, 128), jnp.float32),
               jax.ShapeDtypeStruct((1, 128), jnp.float32),
               jax.ShapeDtypeStruct((1, 128), jnp.float32),
               jax.ShapeDtypeStruct((1, 128), jnp.float32),
               jax.ShapeDtypeStruct((1, 128), jnp.float32)],
)

_mm2 = pl.pallas_call(
    _mm2_body,
    grid=(NBLK,),
    in_specs=[_pk_spec(), _pk_spec(), _pk_spec(),
              _full_spec(1, 128), _full_spec(1, 128),
              _full_spec(1, 128), _full_spec(1, 128),
              _full_spec(128, 128), _full_spec(128, 128),
              _full_spec(128, 128), _full_spec(128, 128)],
    out_specs=[_pk_spec(), _pk_spec()],
    out_shape=[jax.ShapeDtypeStruct((P4, 128), jnp.float32),
               jax.ShapeDtypeStruct((P4, 128), jnp.float32)],
)

_pool = pl.pallas_call(
    _pool_body,
    grid=(NBLK,),
    in_specs=[_pk_spec(), _pk_spec(), _pk_spec(),
              _full_spec(1, 128), _full_spec(1, 128),
              _full_spec(1, 128), _full_spec(1, 128)],
    out_specs=[_full_spec(G, 128)] * 5,
    out_shape=[jax.ShapeDtypeStruct((G, 128), jnp.float32)] * 5,
)

_final = pl.pallas_call(
    _final_body,
    grid=(1,),
    in_specs=[_full_spec(G, 128)] * 5 + [_full_spec(2 * H, 128),
                                         _full_spec(1, 128)],
    out_specs=_full_spec(G, 128),
    out_shape=jax.ShapeDtypeStruct((G, 128), jnp.float32),
)

_deg_kernel = _make_deg_kernel()
_conv_kernel = _make_conv_kernel()


def _blockdiag4(w):
    r, c = w.shape
    z = jnp.zeros((4 * r, 4 * c), jnp.float32)
    for k in range(4):
        z = z.at[k * r:(k + 1) * r, k * c:(k + 1) * c].set(w)
    return z


def _tile4(v):
    return jnp.tile(v.reshape(1, -1), (1, 4))


def _fold128(v):
    return jnp.sum(v.reshape(4, HH), axis=0)


def kernel(x, edge_index, batch, W_embed, b_embed, W1, b1, gamma1, beta1,
           W2, b2, gamma2, beta2, W_out, b_out):
    f32 = jnp.float32
    edge3 = edge_index.astype(jnp.int32).reshape(2, E // CW, CW)
    batch_pad = jnp.pad(batch.astype(jnp.int32), (0, NPAD - N), mode="edge")
    batch4 = jnp.repeat(batch_pad, HH).reshape(P4, 128)
    x4 = x.reshape(N // 4, 4 * D_IN)

    degx_lin = _deg_kernel(edge3)
    degx = degx_lin.reshape(P4, 128)

    ya4, yb4 = _mm1(x4, degx, _blockdiag4(W_embed),
                    _tile4(b_embed), _blockdiag4(W1[:, :HH]),
                    _blockdiag4(W1[:, HH:]))

    def conv_bn(ya4_, yb4_, bvec, gamma, beta):
        sa, sb = _conv_kernel(edge3,
                              ya4_.reshape(NPAD, HH), yb4_.reshape(NPAD, HH))
        ta4, tb4, sma, sqa, smb, sqb = _stats(
            sa.reshape(P4, 128), sb.reshape(P4, 128), degx,
            _tile4(bvec[:HH]), _tile4(bvec[HH:]))
        sm = jnp.concatenate([_fold128(sma), _fold128(smb)])
        sq = jnp.concatenate([_fold128(sqa), _fold128(sqb)])
        mu = sm / N
        var = sq / N - mu * mu
        scale = gamma / jnp.sqrt(var + EPS)
        shift = beta - mu * scale
        return ta4, tb4, scale, shift

    ta4, tb4, scale1, shift1 = conv_bn(ya4, yb4, b1, gamma1, beta1)

    y2a4, y2b4 = _mm2(ta4, tb4, degx,
                      _tile4(scale1[:HH]), _tile4(shift1[:HH]),
                      _tile4(scale1[HH:]), _tile4(shift1[HH:]),
                      _blockdiag4(W2[:HH, :HH]), _blockdiag4(W2[:HH, HH:]),
                      _blockdiag4(W2[HH:, :HH]), _blockdiag4(W2[HH:, HH:]))

    t2a4, t2b4, scale2, shift2 = conv_bn(y2a4, y2b4, b2, gamma2, beta2)

    ma, mb, sua, sub, cnt = _pool(t2a4, t2b4, batch4,
                                  _tile4(scale2[:HH]), _tile4(shift2[:HH]),
                                  _tile4(scale2[HH:]), _tile4(shift2[HH:]))

    return _final(ma, mb, sua, sub, cnt, W_out, b_out.reshape(1, 2 * H))
